# baseline (speedup 1.0000x reference)
import functools

import jax
import jax.numpy as jnp
from jax import lax
from jax.experimental import pallas as pl
from jax.experimental.pallas import tpu as pltpu
from jax.experimental.pallas import tpu_sc as plsc

BATCH, HIST, DIM = 4096, 200, 128
N = BATCH * HIST
TRUMP_SUIT = 3
TRUMP_DELTA = 14
NSUIT = 5
NRANK = 5

NC, NS, L = 2, 16, 16
NW = NC * NS
NHALF = 2
HN = N // NHALF
PER_W = HN // NW
GIDX = 128
BLK = 320
NG = PER_W // BLK
SUIT_BASE = 32

_SCRATCH = [
    pltpu.VMEM((40, DIM), jnp.float32),
    pltpu.VMEM((32, DIM), jnp.float32),
    pltpu.VMEM_SHARED((NS * 32, DIM), jnp.float32),
    pltpu.VMEM((BLK,), jnp.int32),
    pltpu.VMEM((BLK,), jnp.int32),
    pltpu.VMEM((BLK,), jnp.int32),
    pltpu.VMEM((BLK,), jnp.int32),
    pltpu.VMEM((BLK, DIM), jnp.float32),
    pltpu.VMEM((BLK, DIM), jnp.float32),
    pltpu.SemaphoreType.DMA,
    pltpu.SemaphoreType.DMA,
    pltpu.SemaphoreType.DMA,
    pltpu.SemaphoreType.DMA,
    pltpu.SemaphoreType.DMA,
    pltpu.SemaphoreType.DMA,
]


def _make_body(half):
    out_off = half * HN

    def _body(xp_hbm, tab_hbm, out_hbm,
              tab_v, c_v, c_sh, x_v0, x_v1, idx_v0, idx_v1, rows_v0, rows_v1,
              sem_x0, sem_x1, sem_g0, sem_g1, sem_w0, sem_w1):
        cid = lax.axis_index("c")
        sid = lax.axis_index("s")
        wid = sid * NC + cid

        pltpu.sync_copy(tab_hbm.at[pl.ds(0, 40)], tab_v)
        for s in range(NSUIT):
            rbase = 1 + (TRUMP_DELTA if s == TRUMP_SUIT else 0)
            for r in range(NRANK):
                for k in range(DIM // L):
                    c_v[NSUIT * s + r, pl.ds(k * L, L)] = (
                        tab_v[rbase + r, pl.ds(k * L, L)]
                        + tab_v[SUIT_BASE + s + 1, pl.ds(k * L, L)]
                    )
        rep_base = sid * 32
        pltpu.sync_copy(c_v, c_sh.at[pl.ds(rep_base, 32)])

        xbase = wid * PER_W
        obase = out_off + wid * PER_W
        slots = ((x_v0, idx_v0, rows_v0, sem_x0, sem_g0, sem_w0),
                 (x_v1, idx_v1, rows_v1, sem_x1, sem_g1, sem_w1))

        def x_copies(slot, g):
            x_v, _, _, sem_x, _, _ = slots[slot]
            return (
                pltpu.make_async_copy(
                    xp_hbm.at[pl.ds(xbase + g * BLK, BLK)], x_v, sem_x),
            )

        pieces = []
        off = 0
        while off < BLK:
            cnt = min(GIDX, BLK - off)
            pieces.append((off, cnt))
            off += cnt

        def gather_copy(slot, piece):
            _, idx_v, rows_v, _, sem_g, _ = slots[slot]
            p_off, p_cnt = piece
            return pltpu.make_async_copy(
                c_sh.at[idx_v.at[pl.ds(p_off, p_cnt)]],
                rows_v.at[pl.ds(p_off, p_cnt)], sem_g)

        def write_copy(slot, g):
            _, _, rows_v, _, _, sem_w = slots[slot]
            return pltpu.make_async_copy(
                rows_v, out_hbm.at[pl.ds(obase + g * BLK, BLK)], sem_w)

        def do_group(slot, g, first_round, last_round):
            x_v, idx_v, _, _, _, _ = slots[slot]
            for c in x_copies(slot, g):
                c.wait()
            for k in range(BLK // L):
                packed = x_v[pl.ds(k * L, L)]
                rank = packed & 0xFF
                suit = packed >> 8
                idx_v[pl.ds(k * L, L)] = suit * NSUIT + rank + rep_base

            @pl.when(jnp.logical_not(last_round))
            def _prefetch():
                for c in x_copies(slot, g + 2):
                    c.start()

            @pl.when(jnp.logical_not(first_round))
            def _drain_write():
                write_copy(slot, g - 2).wait()

            for piece in pieces:
                gather_copy(slot, piece).start()
            for piece in pieces:
                gather_copy(slot, piece).wait()
            write_copy(slot, g).start()

        for c in x_copies(0, 0) + x_copies(1, 1):
            c.start()

        def pair_body(i, carry):
            g = i * 2
            do_group(0, g, i == 0, i == NG // 2 - 1)
            do_group(1, g + 1, i == 0, i == NG // 2 - 1)
            return carry

        lax.fori_loop(0, NG // 2, pair_body, 0)
        write_copy(0, NG - 2).wait()
        write_copy(1, NG - 1).wait()

    return _body


_MESH = plsc.VectorSubcoreMesh(core_axis_name="c", subcore_axis_name="s")

_sc_half0 = functools.partial(
    pl.kernel,
    mesh=_MESH,
    out_type=jax.ShapeDtypeStruct((N, DIM), jnp.float32),
    scratch_types=_SCRATCH,
    compiler_params=pltpu.CompilerParams(needs_layout_passes=False),
)(_make_body(0))

_sc_half1 = functools.partial(
    pl.kernel,
    mesh=_MESH,
    out_type=(),
    scratch_types=_SCRATCH,
    compiler_params=pltpu.CompilerParams(needs_layout_passes=False),
)(_make_body(1))


def kernel(x, rank_table, suit_table):
    tab = jnp.concatenate([
        jnp.pad(rank_table, ((0, 4), (0, 0))),
        jnp.pad(suit_table, ((0, 9), (0, 0))),
    ])
    hb = BATCH // NHALF
    xp0 = (x[:hb, :, 0] | (x[:hb, :, 1] << 8)).reshape(HN)
    xp1 = (x[hb:, :, 0] | (x[hb:, :, 1] << 8)).reshape(HN)
    out = _sc_half0(xp0, tab)
    out_ref = jax.new_ref(out)
    _sc_half1(xp1, tab, out_ref)
    return jax.freeze(out_ref).reshape(BATCH, HIST, DIM)


# speedup vs baseline: 1.0125x; 1.0125x over previous
import functools

import jax
import jax.numpy as jnp
from jax import lax
from jax.experimental import pallas as pl
from jax.experimental.pallas import tpu as pltpu
from jax.experimental.pallas import tpu_sc as plsc

BATCH, HIST, DIM = 4096, 200, 128
N = BATCH * HIST
TRUMP_SUIT = 3
TRUMP_DELTA = 14
NSUIT = 5
NRANK = 5

NC, NS, L = 2, 16, 16
NW = NC * NS
GIDX = 128
BLK = 320
SUIT_BASE = 32
SPLITS = ((0, 512), (512, 1536), (2048, 2048))

_SCRATCH = [
    pltpu.VMEM((40, DIM), jnp.float32),
    pltpu.VMEM((32, DIM), jnp.float32),
    pltpu.VMEM_SHARED((NS * 32, DIM), jnp.float32),
    pltpu.VMEM((BLK,), jnp.int32),
    pltpu.VMEM((BLK,), jnp.int32),
    pltpu.VMEM((BLK,), jnp.int32),
    pltpu.VMEM((BLK,), jnp.int32),
    pltpu.VMEM((BLK, DIM), jnp.float32),
    pltpu.VMEM((BLK, DIM), jnp.float32),
    pltpu.SemaphoreType.DMA,
    pltpu.SemaphoreType.DMA,
    pltpu.SemaphoreType.DMA,
    pltpu.SemaphoreType.DMA,
    pltpu.SemaphoreType.DMA,
    pltpu.SemaphoreType.DMA,
]


def _make_body(batch_off, batch_cnt):
    out_off = batch_off * HIST
    per_w = batch_cnt * HIST // NW
    ng = per_w // BLK
    assert ng % 2 == 0 and ng * BLK == per_w

    def _body(xp_hbm, tab_hbm, out_hbm,
              tab_v, c_v, c_sh, x_v0, x_v1, idx_v0, idx_v1, rows_v0, rows_v1,
              sem_x0, sem_x1, sem_g0, sem_g1, sem_w0, sem_w1):
        cid = lax.axis_index("c")
        sid = lax.axis_index("s")
        wid = sid * NC + cid

        pltpu.sync_copy(tab_hbm.at[pl.ds(0, 40)], tab_v)
        for s in range(NSUIT):
            rbase = 1 + (TRUMP_DELTA if s == TRUMP_SUIT else 0)
            for r in range(NRANK):
                for k in range(DIM // L):
                    c_v[NSUIT * s + r, pl.ds(k * L, L)] = (
                        tab_v[rbase + r, pl.ds(k * L, L)]
                        + tab_v[SUIT_BASE + s + 1, pl.ds(k * L, L)]
                    )
        rep_base = sid * 32
        pltpu.sync_copy(c_v, c_sh.at[pl.ds(rep_base, 32)])

        xbase = wid * per_w
        obase = out_off + wid * per_w
        slots = ((x_v0, idx_v0, rows_v0, sem_x0, sem_g0, sem_w0),
                 (x_v1, idx_v1, rows_v1, sem_x1, sem_g1, sem_w1))

        def x_copies(slot, g):
            x_v, _, _, sem_x, _, _ = slots[slot]
            return (
                pltpu.make_async_copy(
                    xp_hbm.at[pl.ds(xbase + g * BLK, BLK)], x_v, sem_x),
            )

        pieces = []
        off = 0
        while off < BLK:
            cnt = min(GIDX, BLK - off)
            pieces.append((off, cnt))
            off += cnt

        def gather_copy(slot, piece):
            _, idx_v, rows_v, _, sem_g, _ = slots[slot]
            p_off, p_cnt = piece
            return pltpu.make_async_copy(
                c_sh.at[idx_v.at[pl.ds(p_off, p_cnt)]],
                rows_v.at[pl.ds(p_off, p_cnt)], sem_g)

        def write_copy(slot, g):
            _, _, rows_v, _, _, sem_w = slots[slot]
            return pltpu.make_async_copy(
                rows_v, out_hbm.at[pl.ds(obase + g * BLK, BLK)], sem_w)

        def do_group(slot, g, first_round, last_round):
            x_v, idx_v, _, _, _, _ = slots[slot]
            for c in x_copies(slot, g):
                c.wait()
            for k in range(BLK // L):
                packed = x_v[pl.ds(k * L, L)]
                rank = packed & 0xFF
                suit = packed >> 8
                idx_v[pl.ds(k * L, L)] = suit * NSUIT + rank + rep_base

            @pl.when(jnp.logical_not(last_round))
            def _prefetch():
                for c in x_copies(slot, g + 2):
                    c.start()

            @pl.when(jnp.logical_not(first_round))
            def _drain_write():
                write_copy(slot, g - 2).wait()

            for piece in pieces:
                gather_copy(slot, piece).start()
            for piece in pieces:
                gather_copy(slot, piece).wait()
            write_copy(slot, g).start()

        for c in x_copies(0, 0) + x_copies(1, 1):
            c.start()

        def pair_body(i, carry):
            g = i * 2
            do_group(0, g, i == 0, i == ng // 2 - 1)
            do_group(1, g + 1, i == 0, i == ng // 2 - 1)
            return carry

        lax.fori_loop(0, ng // 2, pair_body, 0)
        write_copy(0, ng - 2).wait()
        write_copy(1, ng - 1).wait()

    return _body


_MESH = plsc.VectorSubcoreMesh(core_axis_name="c", subcore_axis_name="s")

_sc_parts = [
    functools.partial(
        pl.kernel,
        mesh=_MESH,
        out_type=(jax.ShapeDtypeStruct((N, DIM), jnp.float32)
                  if i == 0 else ()),
        scratch_types=_SCRATCH,
        compiler_params=pltpu.CompilerParams(needs_layout_passes=False),
    )(_make_body(boff, bcnt))
    for i, (boff, bcnt) in enumerate(SPLITS)
]


def kernel(x, rank_table, suit_table):
    tab = jnp.concatenate([
        jnp.pad(rank_table, ((0, 4), (0, 0))),
        jnp.pad(suit_table, ((0, 9), (0, 0))),
    ])
    xps = [
        (x[boff:boff + bcnt, :, 0]
         | (x[boff:boff + bcnt, :, 1] << 8)).reshape(bcnt * HIST)
        for boff, bcnt in SPLITS
    ]
    out = _sc_parts[0](xps[0], tab)
    out_ref = jax.new_ref(out)
    for part, xp in zip(_sc_parts[1:], xps[1:]):
        part(xp, tab, out_ref)
    return jax.freeze(out_ref).reshape(BATCH, HIST, DIM)


# speedup vs baseline: 1.0530x; 1.0400x over previous
import functools

import jax
import jax.numpy as jnp
from jax import lax
from jax.experimental import pallas as pl
from jax.experimental.pallas import tpu as pltpu
from jax.experimental.pallas import tpu_sc as plsc

BATCH, HIST, DIM = 4096, 200, 128
N = BATCH * HIST
TRUMP_SUIT = 3
TRUMP_DELTA = 14
NSUIT = 5
NRANK = 5

NC, NS, L = 2, 16, 16
NW = NC * NS
GIDX = 128
BLK = 320
SUIT_BASE = 32
SPLITS = ((0, 512), (512, 3584))

_SCRATCH = [
    pltpu.VMEM((40, DIM), jnp.float32),
    pltpu.VMEM((32, DIM), jnp.float32),
    pltpu.VMEM_SHARED((NS * 32, DIM), jnp.float32),
    pltpu.VMEM((BLK,), jnp.int32),
    pltpu.VMEM((BLK,), jnp.int32),
    pltpu.VMEM((BLK,), jnp.int32),
    pltpu.VMEM((BLK,), jnp.int32),
    pltpu.VMEM((BLK, DIM), jnp.float32),
    pltpu.VMEM((BLK, DIM), jnp.float32),
    pltpu.SemaphoreType.DMA,
    pltpu.SemaphoreType.DMA,
    pltpu.SemaphoreType.DMA,
    pltpu.SemaphoreType.DMA,
    pltpu.SemaphoreType.DMA,
    pltpu.SemaphoreType.DMA,
]


def _make_body(batch_off, batch_cnt):
    out_off = batch_off * HIST
    per_w = batch_cnt * HIST // NW
    ng = per_w // BLK
    assert ng % 2 == 0 and ng * BLK == per_w

    def _body(xp_hbm, tab_hbm, out_hbm,
              tab_v, c_v, c_sh, x_v0, x_v1, idx_v0, idx_v1, rows_v0, rows_v1,
              sem_x0, sem_x1, sem_g0, sem_g1, sem_w0, sem_w1):
        cid = lax.axis_index("c")
        sid = lax.axis_index("s")
        wid = sid * NC + cid

        pltpu.sync_copy(tab_hbm.at[pl.ds(0, 40)], tab_v)
        for s in range(NSUIT):
            rbase = 1 + (TRUMP_DELTA if s == TRUMP_SUIT else 0)
            for r in range(NRANK):
                for k in range(DIM // L):
                    c_v[NSUIT * s + r, pl.ds(k * L, L)] = (
                        tab_v[rbase + r, pl.ds(k * L, L)]
                        + tab_v[SUIT_BASE + s + 1, pl.ds(k * L, L)]
                    )
        rep_base = sid * 32
        pltpu.sync_copy(c_v, c_sh.at[pl.ds(rep_base, 32)])

        xbase = wid * per_w
        obase = out_off + wid * per_w
        slots = ((x_v0, idx_v0, rows_v0, sem_x0, sem_g0, sem_w0),
                 (x_v1, idx_v1, rows_v1, sem_x1, sem_g1, sem_w1))

        def x_copies(slot, g):
            x_v, _, _, sem_x, _, _ = slots[slot]
            return (
                pltpu.make_async_copy(
                    xp_hbm.at[pl.ds(xbase + g * BLK, BLK)], x_v, sem_x),
            )

        pieces = []
        off = 0
        while off < BLK:
            cnt = min(GIDX, BLK - off)
            pieces.append((off, cnt))
            off += cnt

        def gather_copy(slot, piece):
            _, idx_v, rows_v, _, sem_g, _ = slots[slot]
            p_off, p_cnt = piece
            return pltpu.make_async_copy(
                c_sh.at[idx_v.at[pl.ds(p_off, p_cnt)]],
                rows_v.at[pl.ds(p_off, p_cnt)], sem_g)

        def write_copy(slot, g):
            _, _, rows_v, _, _, sem_w = slots[slot]
            return pltpu.make_async_copy(
                rows_v, out_hbm.at[pl.ds(obase + g * BLK, BLK)], sem_w)

        def do_group(slot, g, first_round, last_round):
            x_v, idx_v, _, _, _, _ = slots[slot]
            for c in x_copies(slot, g):
                c.wait()
            for k in range(BLK // L):
                packed = x_v[pl.ds(k * L, L)]
                rank = packed & 0xFF
                suit = packed >> 8
                idx_v[pl.ds(k * L, L)] = suit * NSUIT + rank + rep_base

            @pl.when(jnp.logical_not(last_round))
            def _prefetch():
                for c in x_copies(slot, g + 2):
                    c.start()

            @pl.when(jnp.logical_not(first_round))
            def _drain_write():
                write_copy(slot, g - 2).wait()

            for piece in pieces:
                gather_copy(slot, piece).start()
            for piece in pieces:
                gather_copy(slot, piece).wait()
            write_copy(slot, g).start()

        for c in x_copies(0, 0) + x_copies(1, 1):
            c.start()

        def pair_body(i, carry):
            g = i * 2
            do_group(0, g, i == 0, i == ng // 2 - 1)
            do_group(1, g + 1, i == 0, i == ng // 2 - 1)
            return carry

        lax.fori_loop(0, ng // 2, pair_body, 0)
        write_copy(0, ng - 2).wait()
        write_copy(1, ng - 1).wait()

    return _body


_MESH = plsc.VectorSubcoreMesh(core_axis_name="c", subcore_axis_name="s")

_sc_parts = [
    functools.partial(
        pl.kernel,
        mesh=_MESH,
        out_type=(jax.ShapeDtypeStruct((N, DIM), jnp.float32)
                  if i == 0 else ()),
        scratch_types=_SCRATCH,
        compiler_params=pltpu.CompilerParams(needs_layout_passes=False),
    )(_make_body(boff, bcnt))
    for i, (boff, bcnt) in enumerate(SPLITS)
]


def kernel(x, rank_table, suit_table):
    tab = jnp.concatenate([
        jnp.pad(rank_table, ((0, 4), (0, 0))),
        jnp.pad(suit_table, ((0, 9), (0, 0))),
    ])
    xps = [
        (x[boff:boff + bcnt, :, 0]
         | (x[boff:boff + bcnt, :, 1] << 8)).reshape(bcnt * HIST)
        for boff, bcnt in SPLITS
    ]
    out = _sc_parts[0](xps[0], tab)
    out_ref = jax.new_ref(out)
    for part, xp in zip(_sc_parts[1:], xps[1:]):
        part(xp, tab, out_ref)
    return jax.freeze(out_ref).reshape(BATCH, HIST, DIM)
